# baseline (speedup 1.0000x reference)
import functools

import jax
import jax.numpy as jnp
from jax import lax
from jax.experimental import pallas as pl
from jax.experimental.pallas import tpu as pltpu
from jax.experimental.pallas import tpu_sc as plsc

B = 16384
F = 26
VOCAB = 100000
DIM = 64

NC = 2
NS = 16
NW = NC * NS
ROWS = B * F
RPW = ROWS // NW
CHUNK = 128
NCH = RPW // CHUNK
NBUF = 8
NGRP = NCH // NBUF

_mesh = plsc.VectorSubcoreMesh(core_axis_name="c", subcore_axis_name="s")


@functools.partial(
    pl.kernel,
    out_type=jax.ShapeDtypeStruct((ROWS, DIM), jnp.float32),
    mesh=_mesh,
    scratch_types=[
        pltpu.VMEM((NCH, CHUNK), jnp.int32),
        pltpu.VMEM((NBUF, CHUNK, DIM), jnp.float32),
        pltpu.SemaphoreType.DMA((NBUF,)),
        pltpu.SemaphoreType.DMA((NBUF,)),
    ],
    compiler_params=pltpu.CompilerParams(use_tc_tiling_on_sc=False),
)
def _gather_rows(tables_hbm, idx_hbm, out_hbm, idx_v, bufs, sem_g, sem_w):
    wid = lax.axis_index("s") * NC + lax.axis_index("c")
    row0 = wid * RPW

    pltpu.sync_copy(idx_hbm.at[pl.ds(wid * NCH, NCH)], idx_v)

    def start_gather(c, b):
        pltpu.async_copy(tables_hbm.at[idx_v.at[c]], bufs.at[b], sem_g.at[b])

    def wait_gather(b):
        pltpu.make_async_copy(
            tables_hbm.at[idx_v.at[0]], bufs.at[b], sem_g.at[b]
        ).wait()

    def start_write(c, b):
        pltpu.async_copy(
            bufs.at[b], out_hbm.at[pl.ds(row0 + c * CHUNK, CHUNK)], sem_w.at[b]
        )

    def wait_write(b):
        pltpu.make_async_copy(
            bufs.at[b], out_hbm.at[pl.ds(0, CHUNK)], sem_w.at[b]
        ).wait()

    for b in range(NBUF):
        start_gather(b, b)

    def outer(k, carry):
        for b in range(NBUF):
            wait_gather(b)
            start_write((k - 1) * NBUF + b, b)
        for b in range(NBUF):
            wait_write(b)
            start_gather(k * NBUF + b, b)
        return carry

    lax.fori_loop(1, NGRP, outer, 0)

    for b in range(NBUF):
        wait_gather(b)
        start_write((NGRP - 1) * NBUF + b, b)
    for b in range(NBUF):
        wait_write(b)


def kernel(x_cat, tables):
    x_cat = x_cat.astype(jnp.int32)
    flat_idx = x_cat + (jnp.arange(F, dtype=jnp.int32) * VOCAB)[None, :]
    flat_idx = flat_idx.reshape(ROWS // CHUNK, CHUNK)
    tables_flat = tables.reshape(F * VOCAB, DIM)
    out = _gather_rows(tables_flat, flat_idx)
    return out.reshape(B, F * DIM)


# speedup vs baseline: 1.0089x; 1.0089x over previous
import functools

import jax
import jax.numpy as jnp
from jax import lax
from jax.experimental import pallas as pl
from jax.experimental.pallas import tpu as pltpu
from jax.experimental.pallas import tpu_sc as plsc

B = 16384
F = 26
VOCAB = 100000
DIM = 64

NC = 2
NS = 16
NW = NC * NS
CHUNK = 128
NBLK = B // CHUNK
NPAIR = F // 2
NU = NPAIR * NBLK // NW
NBUF = 2

_mesh = plsc.VectorSubcoreMesh(core_axis_name="c", subcore_axis_name="s")


@functools.partial(
    pl.kernel,
    out_type=jax.ShapeDtypeStruct((B, F * DIM), jnp.float32),
    mesh=_mesh,
    scratch_types=[
        pltpu.VMEM((2 * NU, CHUNK), jnp.int32),
        pltpu.VMEM((NBUF, 2, CHUNK), jnp.int32),
        pltpu.VMEM((NBUF, 2, CHUNK, 2 * DIM), jnp.float32),
        pltpu.VMEM((NBUF, CHUNK, 2 * DIM), jnp.float32),
        pltpu.SemaphoreType.DMA((NBUF,)),
        pltpu.SemaphoreType.DMA((NBUF,)),
    ],
    compiler_params=pltpu.CompilerParams(needs_layout_passes=False),
)
def _gather_rows(tables_hbm, pidx_hbm, csel_hbm, out_hbm,
                 pidx_v, csel_u, bufs, obufs, sem_g, sem_w):
    wid = lax.axis_index("s") * NC + lax.axis_index("c")
    q0 = wid * NU

    pltpu.sync_copy(pidx_hbm.at[pl.ds(2 * q0, 2 * NU)], pidx_v)

    iota16 = lax.iota(jnp.int32, 16)
    zero16 = jnp.zeros((16,), jnp.int32)
    one16 = jnp.ones((16,), jnp.int32)

    def start_gather(u, b):
        pltpu.async_copy(tables_hbm.at[pidx_v.at[2 * u]], bufs.at[b, 0],
                         sem_g.at[b])
        pltpu.async_copy(tables_hbm.at[pidx_v.at[2 * u + 1]], bufs.at[b, 1],
                         sem_g.at[b])
        pltpu.async_copy(csel_hbm.at[pl.ds(2 * (q0 + u), 2)], csel_u.at[b],
                         sem_g.at[b])

    def wait_gather(b):
        for h in range(2):
            pltpu.make_async_copy(
                tables_hbm.at[pidx_v.at[0]], bufs.at[b, h], sem_g.at[b]
            ).wait()
        pltpu.make_async_copy(
            csel_hbm.at[pl.ds(0, 2)], csel_u.at[b], sem_g.at[b]
        ).wait()

    def start_write(u, b):
        q = q0 + u
        fpair = q >> 7
        blk = q & 127
        pltpu.async_copy(
            obufs.at[b],
            out_hbm.at[pl.ds(blk * CHUNK, CHUNK),
                       pl.ds(fpair * 2 * DIM, 2 * DIM)],
            sem_w.at[b],
        )

    def wait_write(b):
        pltpu.make_async_copy(
            obufs.at[b],
            out_hbm.at[pl.ds(0, CHUNK), pl.ds(0, 2 * DIM)],
            sem_w.at[b],
        ).wait()

    def select(b):
        def row_body(r, carry):
            rv = jnp.full((16,), r, jnp.int32)
            ca = plsc.load_gather(csel_u.at[b], [zero16, rv])
            cb = plsc.load_gather(csel_u.at[b], [one16, rv])
            for k in range(DIM // 16):
                va = plsc.load_gather(bufs.at[b, 0], [rv, ca + (iota16 + k * 16)])
                vb = plsc.load_gather(bufs.at[b, 1], [rv, cb + (iota16 + k * 16)])
                obufs[b, r, pl.ds(k * 16, 16)] = va
                obufs[b, r, pl.ds(DIM + k * 16, 16)] = vb
            return carry

        lax.fori_loop(0, CHUNK, row_body, 0, unroll=2)

    for b in range(NBUF):
        start_gather(b, b)
    for b in range(NBUF):
        wait_gather(b)
        select(b)
        start_write(b, b)
        start_gather(NBUF + b, b)

    def outer(k, carry):
        for b in range(NBUF):
            u = k * NBUF + b
            wait_gather(b)
            wait_write(b)
            select(b)
            start_write(u, b)
            start_gather(u + NBUF, b)
        return carry

    lax.fori_loop(1, NU // NBUF - 1, outer, 0)

    for b in range(NBUF):
        u = NU - NBUF + b
        wait_gather(b)
        wait_write(b)
        select(b)
        start_write(u, b)
    for b in range(NBUF):
        wait_write(b)


def kernel(x_cat, tables):
    x_cat = x_cat.astype(jnp.int32)
    flat = x_cat.T + (jnp.arange(F, dtype=jnp.int32) * VOCAB)[:, None]
    flat = flat.reshape(NPAIR, 2, NBLK, CHUNK).transpose(0, 2, 1, 3)
    flat = flat.reshape(2 * NPAIR * NBLK, CHUNK)
    pidx = flat >> 1
    csel = (flat & 1) * DIM
    tables_pair = tables.reshape(F * VOCAB // 2, 2 * DIM)
    return _gather_rows(tables_pair, pidx, csel)
